# Initial kernel scaffold; baseline (speedup 1.0000x reference)
import jax
import jax.numpy as jnp
from jax.experimental import pallas as pl


def kernel(x, edge_index, W1, b1, W2, b2):
    raise NotImplementedError("write your pallas kernel here")


# speedup vs baseline: 13.0152x; 13.0152x over previous
import functools

import jax
import jax.numpy as jnp
from jax import lax
from jax.experimental import pallas as pl
from jax.experimental.pallas import tpu as pltpu
from jax.experimental.pallas import tpu_sc as plsc

N = 10000
D = 128
E = 320000
NC = 2
NS = 16
NW = NC * NS
EPT = E // NW
K = 80
NCHUNK = EPT // K
RPT = 624
TAIL = N - NS * RPT
ZR = 104
HD = 16

_mesh = plsc.VectorSubcoreMesh(core_axis_name="c", subcore_axis_name="s")


@functools.partial(
    pl.kernel,
    out_type=jax.ShapeDtypeStruct((NC * N, HD), jnp.float32),
    mesh=_mesh,
    scratch_types=[
        pltpu.VMEM_SHARED((N, HD), jnp.float32),
        pltpu.VMEM((K,), jnp.int32),
        pltpu.VMEM((K, HD), jnp.float32),
        pltpu.VMEM((RPT, HD), jnp.float32),
    ],
)
def _degree_kernel(dst_hbm, out_hbm, acc_sh, idx_v, ones_v, zero_v):
    core = lax.axis_index("c")
    sub = lax.axis_index("s")

    def fill_ones(i, _):
        ones_v[i, :] = jnp.ones((HD,), jnp.float32)
        zero_v[i, :] = jnp.zeros((HD,), jnp.float32)
        return 0

    lax.fori_loop(0, K, fill_ones, 0)

    def fill_zero(i, _):
        zero_v[i, :] = jnp.zeros((HD,), jnp.float32)
        return 0

    lax.fori_loop(K, RPT, fill_zero, 0)
    pltpu.sync_copy(zero_v, acc_sh.at[pl.ds(sub * RPT, RPT)])

    @pl.when(sub == NS - 1)
    def _zero_tail():
        pltpu.sync_copy(zero_v.at[pl.ds(0, TAIL)], acc_sh.at[pl.ds(NS * RPT, TAIL)])

    plsc.subcore_barrier()

    base = (core * NS + sub) * EPT

    def body(i, _):
        pltpu.sync_copy(dst_hbm.at[pl.ds(base + i * K, K)], idx_v)
        pltpu.sync_copy(ones_v, acc_sh.at[idx_v], add=True)
        return 0

    lax.fori_loop(0, NCHUNK, body, 0)
    plsc.subcore_barrier()
    pltpu.sync_copy(
        acc_sh.at[pl.ds(sub * RPT, RPT)],
        out_hbm.at[pl.ds(core * N + sub * RPT, RPT)],
    )

    @pl.when(sub == NS - 1)
    def _out_tail():
        pltpu.sync_copy(
            acc_sh.at[pl.ds(NS * RPT, TAIL)],
            out_hbm.at[pl.ds(core * N + NS * RPT, TAIL)],
        )


@functools.partial(
    pl.kernel,
    out_type=jax.ShapeDtypeStruct((NC * N, D), jnp.float32),
    mesh=_mesh,
    scratch_types=[
        pltpu.VMEM_SHARED((N, D), jnp.float32),
        pltpu.VMEM((K,), jnp.int32),
        pltpu.VMEM((K,), jnp.int32),
        pltpu.VMEM((K, D), jnp.float32),
        pltpu.VMEM((ZR, D), jnp.float32),
        pltpu.SemaphoreType.DMA,
    ],
)
def _scatter_kernel(g_hbm, src_hbm, dst_hbm, out_hbm,
                    acc_sh, src_v, dst_v, rows_v, zero_v, sem):
    core = lax.axis_index("c")
    sub = lax.axis_index("s")

    def fill_zero(i, _):
        zero_v[i // 8, pl.ds((i % 8) * HD, HD)] = jnp.zeros((HD,), jnp.float32)
        return 0

    lax.fori_loop(0, ZR * (D // HD), fill_zero, 0)
    for q in range(RPT // ZR):
        pltpu.sync_copy(zero_v, acc_sh.at[pl.ds(sub * RPT + q * ZR, ZR)])

    @pl.when(sub == NS - 1)
    def _zero_tail():
        pltpu.sync_copy(zero_v.at[pl.ds(0, TAIL)], acc_sh.at[pl.ds(NS * RPT, TAIL)])

    plsc.subcore_barrier()

    base = (core * NS + sub) * EPT

    def body(i, _):
        off = base + i * K
        pltpu.sync_copy(src_hbm.at[pl.ds(off, K)], src_v)
        pltpu.sync_copy(dst_hbm.at[pl.ds(off, K)], dst_v)
        pltpu.async_copy(g_hbm.at[src_v], rows_v, sem).wait()
        pltpu.sync_copy(rows_v, acc_sh.at[dst_v], add=True)
        return 0

    lax.fori_loop(0, NCHUNK, body, 0)
    plsc.subcore_barrier()
    pltpu.sync_copy(
        acc_sh.at[pl.ds(sub * RPT, RPT)],
        out_hbm.at[pl.ds(core * N + sub * RPT, RPT)],
    )

    @pl.when(sub == NS - 1)
    def _out_tail():
        pltpu.sync_copy(
            acc_sh.at[pl.ds(NS * RPT, TAIL)],
            out_hbm.at[pl.ds(core * N + NS * RPT, TAIL)],
        )


def _tc1_body(x_ref, w1_ref, degp_ref, g1_ref, disb_ref):
    deg = degp_ref[0:N, 0:1] + degp_ref[N:, 0:1] + 1.0
    dis = lax.rsqrt(deg)
    h = jnp.dot(x_ref[...], w1_ref[...],
                preferred_element_type=jnp.float32,
                precision=lax.Precision.HIGHEST)
    g1_ref[...] = h * dis
    disb_ref[...] = jnp.broadcast_to(dis, (N, D))


def _tc2_body(aggp_ref, g1_ref, disb_ref, w2_ref, b1_ref, g2_ref):
    agg = aggp_ref[0:N, :] + aggp_ref[N:, :] + g1_ref[...]
    t = jnp.maximum(disb_ref[...] * agg + b1_ref[...], 0.0)
    h2 = jnp.dot(t, w2_ref[...],
                 preferred_element_type=jnp.float32,
                 precision=lax.Precision.HIGHEST)
    g2_ref[...] = disb_ref[...] * h2


def _tc3_body(aggp_ref, g2_ref, disb_ref, b2_ref, out_ref):
    agg = aggp_ref[0:N, :] + aggp_ref[N:, :] + g2_ref[...]
    out_ref[...] = disb_ref[...] * agg + b2_ref[...]


def kernel(x, edge_index, W1, b1, W2, b2):
    src = edge_index[0].astype(jnp.int32)
    dst = edge_index[1].astype(jnp.int32)

    degp = _degree_kernel(dst)

    g1, disb = pl.pallas_call(
        _tc1_body,
        out_shape=(
            jax.ShapeDtypeStruct((N, D), jnp.float32),
            jax.ShapeDtypeStruct((N, D), jnp.float32),
        ),
    )(x, W1, degp)

    aggp1 = _scatter_kernel(g1, src, dst)

    g2 = pl.pallas_call(
        _tc2_body,
        out_shape=jax.ShapeDtypeStruct((N, D), jnp.float32),
    )(aggp1, g1, disb, W2, b1.reshape(1, D))

    aggp2 = _scatter_kernel(g2, src, dst)

    out = pl.pallas_call(
        _tc3_body,
        out_shape=jax.ShapeDtypeStruct((N, D), jnp.float32),
    )(aggp2, g2, disb, b2.reshape(1, D))

    return out
